# baseline (speedup 1.0000x reference)
import functools

import jax
import jax.numpy as jnp
from jax import lax
from jax.experimental import pallas as pl
from jax.experimental.pallas import tpu as pltpu
from jax.experimental.pallas import tpu_sc as plsc

_N = 5000
_B = 128
_NB = 40
_NPAD = _NB * _B
_T = 0.5

_NC = 2
_NS = 16
_NW = _NC * _NS
_L = 16
_CHUNK = _NPAD // _NW


def _row2col(row, eye):
    return jnp.sum(row * eye, axis=1, keepdims=True)


def _col2row(col, eye):
    return jnp.sum(col * eye, axis=0, keepdims=True)


def _iou_tile(x1c, y1c, x2c, y2c, ac, x1r, y1r, x2r, y2r, ar):
    xx1 = jnp.maximum(x1c, x1r)
    yy1 = jnp.maximum(y1c, y1r)
    xx2 = jnp.minimum(x2c, x2r)
    yy2 = jnp.minimum(y2c, y2r)
    w = jnp.maximum(xx2 - xx1, 0.0)
    h = jnp.maximum(yy2 - yy1, 0.0)
    inter = w * h
    union = ac + ar - inter
    return inter / (union + 1e-9)


def _rank_kernel(s_ref, s3_ref, rank3_ref):
    f32 = jnp.float32
    i32 = jnp.int32
    sub = jax.lax.broadcasted_iota(i32, (_B, _B), 0)
    lane = jax.lax.broadcasted_iota(i32, (_B, _B), 1)
    gtmask = (sub > lane).astype(f32)

    def rank_block(g, _):
        srow_g = s_ref[pl.ds(g, 1), :]
        scol_g = s3_ref[pl.ds(g, 1), :, :].reshape(_B, 1)

        gq = g // 4

        def chunk_ge(t, accs):
            a0, a1 = accs
            rows = s_ref[pl.ds(t * 4, 4), :]
            a0 = a0 + (rows[0:1, :] >= scol_g).astype(f32)
            a1 = a1 + (rows[1:2, :] >= scol_g).astype(f32)
            a0 = a0 + (rows[2:3, :] >= scol_g).astype(f32)
            a1 = a1 + (rows[3:4, :] >= scol_g).astype(f32)
            return a0, a1

        def chunk_gt(t, accs):
            a0, a1 = accs
            rows = s_ref[pl.ds(t * 4, 4), :]
            a0 = a0 + (rows[0:1, :] > scol_g).astype(f32)
            a1 = a1 + (rows[1:2, :] > scol_g).astype(f32)
            a0 = a0 + (rows[2:3, :] > scol_g).astype(f32)
            a1 = a1 + (rows[3:4, :] > scol_g).astype(f32)
            return a0, a1

        z = jnp.zeros((_B, _B), f32)
        a0, a1 = jax.lax.fori_loop(0, gq, chunk_ge, (z, z))
        a0, a1 = jax.lax.fori_loop(gq + 1, _NB // 4, chunk_gt, (a0, a1))
        rows = s_ref[pl.ds(gq * 4, 4), :]
        for u in range(4):
            r1 = rows[u:u + 1, :]
            gt = (r1 > scol_g).astype(f32)
            eq = (r1 == scol_g).astype(f32)
            w = ((gq * 4 + u) < g).astype(f32)
            a0 = a0 + gt + eq * w
        eq_gg = (srow_g == scol_g).astype(f32)
        accm = a0 + a1 + eq_gg * gtmask
        rank_col = jnp.sum(accm, axis=1, keepdims=True)
        rank3_ref[pl.ds(g, 1), :, :] = rank_col.reshape(1, _B, 1).astype(i32)
        return 0

    jax.lax.fori_loop(0, _NB, rank_block, 0)


def _nms_kernel(x1_ref, y1_ref, x2_ref, y2_ref,
                x13_ref, y13_ref, x23_ref, y23_ref, keep_ref):
    f32 = jnp.float32
    i32 = jnp.int32
    sub = jax.lax.broadcasted_iota(i32, (_B, _B), 0)
    lane = jax.lax.broadcasted_iota(i32, (_B, _B), 1)
    eye = (sub == lane).astype(f32)
    ltmask = (sub < lane).astype(f32)
    pagei4 = jax.lax.broadcasted_iota(i32, (4, 1), 0)

    keep_ref[...] = jnp.ones((_NB, _B), f32)

    def nms_block(b, _):
        x1r = x1_ref[pl.ds(b, 1), :]
        y1r = y1_ref[pl.ds(b, 1), :]
        x2r = x2_ref[pl.ds(b, 1), :]
        y2r = y2_ref[pl.ds(b, 1), :]
        ar = (x2r - x1r) * (y2r - y1r)
        x1c = x13_ref[pl.ds(b, 1), :, :].reshape(_B, 1)
        y1c = y13_ref[pl.ds(b, 1), :, :].reshape(_B, 1)
        x2c = x23_ref[pl.ds(b, 1), :, :].reshape(_B, 1)
        y2c = y23_ref[pl.ds(b, 1), :, :].reshape(_B, 1)
        ac = (x2c - x1c) * (y2c - y1c)

        iou_bb = _iou_tile(x1c, y1c, x2c, y2c, ac, x1r, y1r, x2r, y2r, ar)
        s_intra = (iou_bb > _T).astype(f32) * ltmask

        ext_row = keep_ref[pl.ds(b, 1), :]

        def relax_cond(carry):
            _, go = carry
            return go

        def relax_body(carry):
            krow, _ = carry
            kcol = _row2col(krow, eye)
            killed = jnp.max(s_intra * kcol, axis=0, keepdims=True)
            new = ext_row * (1.0 - killed)
            go = jnp.any(new != krow)
            return new, go

        krow, _ = jax.lax.while_loop(relax_cond, relax_body,
                                     (ext_row, jnp.bool_(True)))
        keep_ref[pl.ds(b, 1), :] = krow
        kcol = _row2col(krow, eye)

        def push4(t, _):
            c0 = t * 4
            cx1_4 = x1_ref[pl.ds(c0, 4), :]
            cy1_4 = y1_ref[pl.ds(c0, 4), :]
            cx2_4 = x2_ref[pl.ds(c0, 4), :]
            cy2_4 = y2_ref[pl.ds(c0, 4), :]
            killed_rows = []
            for u in range(4):
                cx1 = cx1_4[u:u + 1, :]
                cy1 = cy1_4[u:u + 1, :]
                cx2 = cx2_4[u:u + 1, :]
                cy2 = cy2_4[u:u + 1, :]
                car = (cx2 - cx1) * (cy2 - cy1)
                iou_bc = _iou_tile(x1c, y1c, x2c, y2c, ac,
                                   cx1, cy1, cx2, cy2, car)
                killed_rows.append(
                    (jnp.max(iou_bc * kcol, axis=0, keepdims=True)
                     > _T).astype(f32))
            killed4 = jnp.concatenate(killed_rows, axis=0)
            rowmask = ((c0 + pagei4) > b).astype(f32)
            keep_ref[pl.ds(c0, 4), :] = (
                keep_ref[pl.ds(c0, 4), :] * (1.0 - killed4 * rowmask))
            return 0

        jax.lax.fori_loop((b + 1) // 4, _NB // 4, push4, 0)
        return 0

    jax.lax.fori_loop(0, _NB, nms_block, 0)


@functools.lru_cache(maxsize=None)
def _sc_kernels():
    mesh = plsc.VectorSubcoreMesh(core_axis_name="c", subcore_axis_name="s",
                                  num_cores=_NC, num_subcores=_NS)

    @functools.partial(
        pl.kernel,
        out_type=[jax.ShapeDtypeStruct((_NPAD,), jnp.float32)] * 4,
        mesh=mesh,
        compiler_params=pltpu.CompilerParams(needs_layout_passes=False),
        scratch_types=[pltpu.VMEM((_NPAD,), jnp.float32)] * 4
        + [pltpu.VMEM((_NPAD,), jnp.int32),
           pltpu.VMEM((_NPAD,), jnp.int32),
           pltpu.VMEM((_CHUNK,), jnp.float32)],
    )
    def sc_sort_gather(x1h, y1h, x2h, y2h, rankh,
                       ox1, oy1, ox2, oy2,
                       x1v, y1v, x2v, y2v, rankv, invv, outv):
        wid = lax.axis_index("s") * _NC + lax.axis_index("c")
        base = wid * _CHUNK
        pltpu.sync_copy(rankh, rankv)
        pltpu.sync_copy(x1h, x1v)
        pltpu.sync_copy(y1h, y1v)
        pltpu.sync_copy(x2h, x2v)
        pltpu.sync_copy(y2h, y2v)

        lane = lax.iota(jnp.int32, _L)

        def inv_body(j, _):
            idx = rankv[pl.ds(j * _L, _L)]
            plsc.store_scatter(invv, [idx], j * _L + lane)
            return 0

        lax.fori_loop(0, _NPAD // _L, inv_body, 0)

        for src, dst in ((x1v, ox1), (y1v, oy1), (x2v, ox2), (y2v, oy2)):
            for j in range(_CHUNK // _L):
                idx = invv[pl.ds(base + j * _L, _L)]
                outv[pl.ds(j * _L, _L)] = plsc.load_gather(src, [idx])
            pltpu.sync_copy(outv, dst.at[pl.ds(base, _CHUNK)])

    @functools.partial(
        pl.kernel,
        out_type=jax.ShapeDtypeStruct((_NPAD,), jnp.float32),
        mesh=mesh,
        compiler_params=pltpu.CompilerParams(needs_layout_passes=False),
        scratch_types=[pltpu.VMEM((_NPAD,), jnp.float32),
                       pltpu.VMEM((_CHUNK,), jnp.int32),
                       pltpu.VMEM((_CHUNK,), jnp.float32),
                       pltpu.VMEM((_CHUNK,), jnp.float32)],
    )
    def sc_unsort_gather(keeph, rankh, sh, outh, keepv, rankv, sv, outv):
        wid = lax.axis_index("s") * _NC + lax.axis_index("c")
        base = wid * _CHUNK
        pltpu.sync_copy(keeph, keepv)
        pltpu.sync_copy(rankh.at[pl.ds(base, _CHUNK)], rankv)
        pltpu.sync_copy(sh.at[pl.ds(base, _CHUNK)], sv)
        for j in range(_CHUNK // _L):
            idx = rankv[pl.ds(j * _L, _L)]
            k = plsc.load_gather(keepv, [idx])
            outv[pl.ds(j * _L, _L)] = k * sv[pl.ds(j * _L, _L)]
        pltpu.sync_copy(outv, outh.at[pl.ds(base, _CHUNK)])

    return sc_sort_gather, sc_unsort_gather


def kernel(boxes, scores):
    pad = _NPAD - _N
    x1 = jnp.pad(boxes[:, 0], (0, pad))
    y1 = jnp.pad(boxes[:, 1], (0, pad))
    x2 = jnp.pad(boxes[:, 2], (0, pad))
    y2 = jnp.pad(boxes[:, 3], (0, pad))
    s = jnp.pad(scores, (0, pad), constant_values=-1.0)
    s2d = s.reshape(_NB, _B)

    rank3 = pl.pallas_call(
        _rank_kernel,
        out_shape=jax.ShapeDtypeStruct((_NB, _B, 1), jnp.int32),
    )(s2d, s2d.reshape(_NB, _B, 1))

    sc_sort_gather, sc_unsort_gather = _sc_kernels()
    rankf = rank3.reshape(-1)
    sx1, sy1, sx2, sy2 = sc_sort_gather(x1, y1, x2, y2, rankf)

    keep2d = pl.pallas_call(
        _nms_kernel,
        out_shape=jax.ShapeDtypeStruct((_NB, _B), jnp.float32),
    )(sx1.reshape(_NB, _B), sy1.reshape(_NB, _B),
      sx2.reshape(_NB, _B), sy2.reshape(_NB, _B),
      sx1.reshape(_NB, _B, 1), sy1.reshape(_NB, _B, 1),
      sx2.reshape(_NB, _B, 1), sy2.reshape(_NB, _B, 1))

    out = sc_unsort_gather(keep2d.reshape(-1), rankf, s)
    return out[:_N]


# speedup vs baseline: 1.3788x; 1.3788x over previous
import functools

import jax
import jax.numpy as jnp
from jax import lax
from jax.experimental import pallas as pl
from jax.experimental.pallas import tpu as pltpu
from jax.experimental.pallas import tpu_sc as plsc

_N = 5000
_B = 128
_NB = 40
_NPAD = _NB * _B
_T = 0.5

_NC = 2
_NS = 16
_NW = _NC * _NS
_L = 16
_CHUNK = _NPAD // _NW


def _row2col(row, eye):
    return jnp.sum(row * eye, axis=1, keepdims=True)


def _col2row(col, eye):
    return jnp.sum(col * eye, axis=0, keepdims=True)


def _iou_tile(x1c, y1c, x2c, y2c, ac, x1r, y1r, x2r, y2r, ar):
    xx1 = jnp.maximum(x1c, x1r)
    yy1 = jnp.maximum(y1c, y1r)
    xx2 = jnp.minimum(x2c, x2r)
    yy2 = jnp.minimum(y2c, y2r)
    w = jnp.maximum(xx2 - xx1, 0.0)
    h = jnp.maximum(yy2 - yy1, 0.0)
    inter = w * h
    union = ac + ar - inter
    return inter / (union + 1e-9)


def _rank_kernel(s_ref, rank_ref):
    f32 = jnp.float32
    i32 = jnp.int32
    sub = jax.lax.broadcasted_iota(i32, (_B, _B), 0)
    lane = jax.lax.broadcasted_iota(i32, (_B, _B), 1)
    eye = (sub == lane).astype(f32)
    gtmask = (sub > lane).astype(f32)

    def rank_block(g, _):
        srow_g = s_ref[pl.ds(g, 1), :]
        scol_g = _row2col(srow_g, eye)

        gq = g // 4

        def chunk_ge(t, accs):
            a0, a1 = accs
            rows = s_ref[pl.ds(t * 4, 4), :]
            a0 = a0 + (rows[0:1, :] >= scol_g).astype(f32)
            a1 = a1 + (rows[1:2, :] >= scol_g).astype(f32)
            a0 = a0 + (rows[2:3, :] >= scol_g).astype(f32)
            a1 = a1 + (rows[3:4, :] >= scol_g).astype(f32)
            return a0, a1

        def chunk_gt(t, accs):
            a0, a1 = accs
            rows = s_ref[pl.ds(t * 4, 4), :]
            a0 = a0 + (rows[0:1, :] > scol_g).astype(f32)
            a1 = a1 + (rows[1:2, :] > scol_g).astype(f32)
            a0 = a0 + (rows[2:3, :] > scol_g).astype(f32)
            a1 = a1 + (rows[3:4, :] > scol_g).astype(f32)
            return a0, a1

        z = jnp.zeros((_B, _B), f32)
        a0, a1 = jax.lax.fori_loop(0, gq, chunk_ge, (z, z))
        a0, a1 = jax.lax.fori_loop(gq + 1, _NB // 4, chunk_gt, (a0, a1))
        rows = s_ref[pl.ds(gq * 4, 4), :]
        for u in range(4):
            r1 = rows[u:u + 1, :]
            gt = (r1 > scol_g).astype(f32)
            eq = (r1 == scol_g).astype(f32)
            w = ((gq * 4 + u) < g).astype(f32)
            a0 = a0 + gt + eq * w
        eq_gg = (srow_g == scol_g).astype(f32)
        accm = a0 + a1 + eq_gg * gtmask
        rank_col = jnp.sum(accm, axis=1, keepdims=True)
        rank_ref[pl.ds(g, 1), :] = _col2row(rank_col, eye).astype(i32)
        return 0

    jax.lax.fori_loop(0, _NB, rank_block, 0)


def _nms_kernel(x1_ref, y1_ref, x2_ref, y2_ref, keep_ref):
    f32 = jnp.float32
    i32 = jnp.int32
    sub = jax.lax.broadcasted_iota(i32, (_B, _B), 0)
    lane = jax.lax.broadcasted_iota(i32, (_B, _B), 1)
    eye = (sub == lane).astype(f32)
    ltmask = (sub < lane).astype(f32)
    pagei4 = jax.lax.broadcasted_iota(i32, (4, 1), 0)

    keep_ref[...] = jnp.ones((_NB, _B), f32)

    def nms_block(b, _):
        x1r = x1_ref[pl.ds(b, 1), :]
        y1r = y1_ref[pl.ds(b, 1), :]
        x2r = x2_ref[pl.ds(b, 1), :]
        y2r = y2_ref[pl.ds(b, 1), :]
        ar = (x2r - x1r) * (y2r - y1r)
        x1c = _row2col(x1r, eye)
        y1c = _row2col(y1r, eye)
        x2c = _row2col(x2r, eye)
        y2c = _row2col(y2r, eye)
        ac = _row2col(ar, eye)

        iou_bb = _iou_tile(x1c, y1c, x2c, y2c, ac, x1r, y1r, x2r, y2r, ar)
        s_intra = (iou_bb > _T).astype(f32) * ltmask

        ext_row = keep_ref[pl.ds(b, 1), :]

        def relax_cond(carry):
            _, go = carry
            return go

        def relax_body(carry):
            krow, _ = carry
            kcol = _row2col(krow, eye)
            killed = jnp.max(s_intra * kcol, axis=0, keepdims=True)
            new = ext_row * (1.0 - killed)
            go = jnp.any(new != krow)
            return new, go

        krow, _ = jax.lax.while_loop(relax_cond, relax_body,
                                     (ext_row, jnp.bool_(True)))
        keep_ref[pl.ds(b, 1), :] = krow
        kcol = _row2col(krow, eye)

        def push4(t, _):
            c0 = t * 4
            cx1_4 = x1_ref[pl.ds(c0, 4), :]
            cy1_4 = y1_ref[pl.ds(c0, 4), :]
            cx2_4 = x2_ref[pl.ds(c0, 4), :]
            cy2_4 = y2_ref[pl.ds(c0, 4), :]
            killed_rows = []
            for u in range(4):
                cx1 = cx1_4[u:u + 1, :]
                cy1 = cy1_4[u:u + 1, :]
                cx2 = cx2_4[u:u + 1, :]
                cy2 = cy2_4[u:u + 1, :]
                car = (cx2 - cx1) * (cy2 - cy1)
                iou_bc = _iou_tile(x1c, y1c, x2c, y2c, ac,
                                   cx1, cy1, cx2, cy2, car)
                killed_rows.append(
                    (jnp.max(iou_bc * kcol, axis=0, keepdims=True)
                     > _T).astype(f32))
            killed4 = jnp.concatenate(killed_rows, axis=0)
            rowmask = ((c0 + pagei4) > b).astype(f32)
            keep_ref[pl.ds(c0, 4), :] = (
                keep_ref[pl.ds(c0, 4), :] * (1.0 - killed4 * rowmask))
            return 0

        jax.lax.fori_loop((b + 1) // 4, _NB // 4, push4, 0)
        return 0

    jax.lax.fori_loop(0, _NB, nms_block, 0)


@functools.lru_cache(maxsize=None)
def _sc_kernels():
    mesh = plsc.VectorSubcoreMesh(core_axis_name="c", subcore_axis_name="s",
                                  num_cores=_NC, num_subcores=_NS)

    @functools.partial(
        pl.kernel,
        out_type=[jax.ShapeDtypeStruct((_NPAD,), jnp.float32)] * 4,
        mesh=mesh,
        compiler_params=pltpu.CompilerParams(needs_layout_passes=False),
        scratch_types=[pltpu.VMEM((_N * 4,), jnp.float32),
                       pltpu.VMEM((_NPAD,), jnp.int32),
                       pltpu.VMEM((_NPAD,), jnp.int32),
                       pltpu.VMEM((_CHUNK,), jnp.float32)],
    )
    def sc_sort_gather(boxesh, rankh,
                       ox1, oy1, ox2, oy2,
                       boxesv, rankv, invv, outv):
        wid = lax.axis_index("s") * _NC + lax.axis_index("c")
        base = wid * _CHUNK
        pltpu.sync_copy(rankh, rankv)
        pltpu.sync_copy(boxesh, boxesv)

        lane = lax.iota(jnp.int32, _L)

        def inv_body(j, _):
            idx = rankv[pl.ds(j * _L, _L)]
            plsc.store_scatter(invv, [idx], j * _L + lane)
            return 0

        lax.fori_loop(0, _NPAD // _L, inv_body, 0)

        nmax = jnp.full((_L,), _N - 1, jnp.int32)
        for d, dst in enumerate((ox1, oy1, ox2, oy2)):
            for j in range(_CHUNK // _L):
                idx = jnp.minimum(invv[pl.ds(base + j * _L, _L)], nmax)
                outv[pl.ds(j * _L, _L)] = plsc.load_gather(
                    boxesv, [idx * 4 + d])
            pltpu.sync_copy(outv, dst.at[pl.ds(base, _CHUNK)])

    @functools.partial(
        pl.kernel,
        out_type=jax.ShapeDtypeStruct((_NPAD,), jnp.float32),
        mesh=mesh,
        compiler_params=pltpu.CompilerParams(needs_layout_passes=False),
        scratch_types=[pltpu.VMEM((_NPAD,), jnp.float32),
                       pltpu.VMEM((_CHUNK,), jnp.int32),
                       pltpu.VMEM((_CHUNK,), jnp.float32),
                       pltpu.VMEM((_CHUNK,), jnp.float32)],
    )
    def sc_unsort_gather(keeph, rankh, sh, outh, keepv, rankv, sv, outv):
        wid = lax.axis_index("s") * _NC + lax.axis_index("c")
        base = wid * _CHUNK
        pltpu.sync_copy(keeph, keepv)
        pltpu.sync_copy(rankh.at[pl.ds(base, _CHUNK)], rankv)
        pltpu.sync_copy(sh.at[pl.ds(base, _CHUNK)], sv)
        for j in range(_CHUNK // _L):
            idx = rankv[pl.ds(j * _L, _L)]
            k = plsc.load_gather(keepv, [idx])
            outv[pl.ds(j * _L, _L)] = k * sv[pl.ds(j * _L, _L)]
        pltpu.sync_copy(outv, outh.at[pl.ds(base, _CHUNK)])

    return sc_sort_gather, sc_unsort_gather


def kernel(boxes, scores):
    pad = _NPAD - _N
    s = jnp.pad(scores, (0, pad), constant_values=-1.0)
    s2d = s.reshape(_NB, _B)

    rank2d = pl.pallas_call(
        _rank_kernel,
        out_shape=jax.ShapeDtypeStruct((_NB, _B), jnp.int32),
    )(s2d)

    sc_sort_gather, sc_unsort_gather = _sc_kernels()
    sx1, sy1, sx2, sy2 = sc_sort_gather(boxes.reshape(-1),
                                        rank2d.reshape(-1))

    keep2d = pl.pallas_call(
        _nms_kernel,
        out_shape=jax.ShapeDtypeStruct((_NB, _B), jnp.float32),
    )(sx1.reshape(_NB, _B), sy1.reshape(_NB, _B),
      sx2.reshape(_NB, _B), sy2.reshape(_NB, _B))

    out = sc_unsort_gather(keep2d.reshape(-1), rank2d.reshape(-1), s)
    return out[:_N]
